# Initial kernel scaffold; baseline (speedup 1.0000x reference)
import jax
import jax.numpy as jnp
from jax.experimental import pallas as pl


def kernel(atom_list, bond_list, atom_degree_list, bond_degree_list, atom_mask, params):
    raise NotImplementedError("write your pallas kernel here")


# speedup vs baseline: 11.9215x; 11.9215x over previous
import functools

import jax
import jax.numpy as jnp
from jax.experimental import pallas as pl

RADIUS = 2
T = 2
FDIM = 39
BDIM = 10
FP = 64
B = 1024
MOL = 128
NB = 6

BS = 8

_HI = jax.lax.Precision.HIGHEST


def _dotT(x, w):
    return jax.lax.dot_general(x, w, (((1,), (1,)), ((), ())),
                               precision=_HI, preferred_element_type=jnp.float32)


def _dot(x, w):
    return jax.lax.dot_general(x, w, (((1,), (0,)), ((), ())),
                               precision=_HI, preferred_element_type=jnp.float32)


def _seg_sum(e_mat, x):
    return jax.lax.dot_general(e_mat, x, (((0,), (0,)), ((), ())),
                               precision=_HI, preferred_element_type=jnp.float32)


def _leaky(x):
    return jnp.where(x >= 0, x, 0.01 * x)


def _elu(x):
    return jnp.where(x > 0, x, jnp.exp(jnp.minimum(x, 0.0)) - 1.0)


def _gru(x, h, wih, whh, bih, bhh):
    gi = _dotT(x, wih) + bih
    gh = _dotT(h, whh) + bhh
    r = jax.nn.sigmoid(gi[:, 0:FP] + gh[:, 0:FP])
    z = jax.nn.sigmoid(gi[:, FP:2 * FP] + gh[:, FP:2 * FP])
    n = jnp.tanh(gi[:, 2 * FP:] + r * gh[:, 2 * FP:])
    return (1.0 - z) * n + z * h


def _gather_block(table, idx, j):
    iota = jax.lax.broadcasted_iota(jnp.int32, (MOL, MOL), 1)
    pieces = []
    for m in range(BS):
        col = idx[m * MOL:(m + 1) * MOL, j:j + 1]
        onehot = (col == iota).astype(jnp.float32)
        pieces.append(_dot(onehot, table[m * MOL:(m + 1) * MOL, :]))
    return jnp.concatenate(pieces, axis=0)


def _attention_round(af, nf_list, idxA, alw, alb, atw, atb):
    wa = alw[:, 0:FP]
    wn = alw[:, FP:2 * FP]
    base = jnp.sum(af * wa, axis=1, keepdims=True) + alb
    scores = []
    amasks = []
    for j in range(NB):
        pad = idxA[:, j:j + 1] == (MOL - 1)
        smask = jnp.where(pad, -9e8, 0.0).astype(jnp.float32)
        amasks.append(jnp.where(pad, 0.0, 1.0).astype(jnp.float32))
        scores.append(_leaky(base + jnp.sum(nf_list[j] * wn, axis=1,
                                            keepdims=True)) + smask)
    mx = scores[0]
    for j in range(1, NB):
        mx = jnp.maximum(mx, scores[j])
    es = [jnp.exp(s - mx) for s in scores]
    denom = es[0]
    for j in range(1, NB):
        denom = denom + es[j]
    ctx = jnp.zeros_like(af)
    for j in range(NB):
        aw = es[j] / denom * amasks[j]
        ctx = ctx + aw * (_dotT(nf_list[j], atw) + atb)
    return _elu(ctx)


def _fp_kernel(atom_ref, bond_ref, idxA_ref, idxB_ref, mask_ref,
               afc_w_ref, afc_b_ref, nfc_w_ref, nfc_b_ref,
               gwi_ref, gbi_ref, gwh_ref, gbh_ref,
               alw_ref, alb_ref, atw_ref, atb_ref,
               mgwi_ref, mgbi_ref, mgwh_ref, mgbh_ref,
               malw_ref, malb_ref, matw_ref, matb_ref,
               ow_ref, ob_ref,
               out_af_ref, out_pred_ref):
    R = BS * MOL
    atom = atom_ref[...]
    bond = bond_ref[...]
    idxA = idxA_ref[...]
    idxB = idxB_ref[...]
    mask = mask_ref[...]

    af = _leaky(_dotT(atom, afc_w_ref[...]) + afc_b_ref[...])

    nfc_w = nfc_w_ref[...]
    wA = nfc_w[:, 0:FDIM]
    wB = nfc_w[:, FDIM:FDIM + BDIM]
    nf0 = []
    for j in range(NB):
        ga = _gather_block(atom, idxA, j)
        gb = _gather_block(bond, idxB, j)
        nf0.append(_leaky(_dotT(ga, wA) + _dotT(gb, wB) + nfc_b_ref[...]))

    h = af
    for d in range(RADIUS):
        if d == 0:
            nf = nf0
            cur = af
        else:
            nf = [_gather_block(act, idxA, j) for j in range(NB)]
            cur = act
        ctx = _attention_round(
            cur, nf, idxA,
            alw_ref[d], alb_ref[d], atw_ref[d], atb_ref[d])
        h = _gru(ctx, h, gwi_ref[d], gwh_ref[d], gbi_ref[d], gbh_ref[d])
        act = jnp.maximum(h, 0.0)

    out_af_ref[...] = h

    e_mat = (jax.lax.broadcasted_iota(jnp.int32, (R, BS), 0) // MOL ==
             jax.lax.broadcasted_iota(jnp.int32, (R, BS), 1)).astype(jnp.float32)
    mol_feature = _seg_sum(e_mat, act * mask)
    act_mol = jnp.maximum(mol_feature, 0.0)
    mol_smask = jnp.where(mask == 0.0, -9e8, 0.0).astype(jnp.float32)

    malw = malw_ref[...]
    w_mol = malw[:, 0:FP]
    w_atom = malw[:, FP:2 * FP]
    aft = _dotT(act, matw_ref[...]) + matb_ref[...]
    for _ in range(T):
        act_mol_rows = _dot(e_mat, act_mol)
        mas = _leaky(jnp.sum(act_mol_rows * w_mol, axis=1, keepdims=True) +
                     jnp.sum(act * w_atom, axis=1, keepdims=True) +
                     malb_ref[...]) + mol_smask
        shifted = []
        for m in range(BS):
            sl = mas[m * MOL:(m + 1) * MOL, :]
            shifted.append(sl - jnp.max(sl))
        e = jnp.exp(jnp.concatenate(shifted, axis=0)) * mask
        num = _seg_sum(e_mat, e * aft)
        den = _seg_sum(e_mat, jnp.broadcast_to(e, (R, FP)))
        mc = _elu(num / den)
        mol_feature = _gru(mc, mol_feature, mgwi_ref[...], mgwh_ref[...],
                           mgbi_ref[...], mgbh_ref[...])
        act_mol = jnp.maximum(mol_feature, 0.0)

    out_pred_ref[...] = (jnp.sum(mol_feature * ow_ref[...], axis=1,
                                 keepdims=True) + ob_ref[...])


@functools.partial(jax.jit, static_argnames=("interpret",))
def _fingerprint(atom_list, bond_list, atom_degree_list, bond_degree_list,
                 atom_mask, params, interpret=False):
    b, mol, fdim = atom_list.shape
    assert mol == MOL and fdim == FDIM
    R = BS * MOL
    n_blocks = b // BS

    atom2d = atom_list.reshape(b * mol, FDIM)
    bond2d = bond_list.reshape(b * mol, BDIM)
    idxA2d = atom_degree_list.reshape(b * mol, NB).astype(jnp.int32)
    idxB2d = bond_degree_list.reshape(b * mol, NB).astype(jnp.int32)
    mask2d = atom_mask.reshape(b * mol, 1)

    p = params
    row = lambda v: v.reshape(1, -1)
    ins = [
        atom2d, bond2d, idxA2d, idxB2d, mask2d,
        p['atom_fc_w'], row(p['atom_fc_b']),
        p['neighbor_fc_w'], row(p['neighbor_fc_b']),
        p['gru_wih'], p['gru_bih'].reshape(RADIUS, 1, 3 * FP),
        p['gru_whh'], p['gru_bhh'].reshape(RADIUS, 1, 3 * FP),
        p['align_w'], p['align_b'].reshape(RADIUS, 1, 1),
        p['attend_w'], p['attend_b'].reshape(RADIUS, 1, FP),
        p['mol_gru_wih'], row(p['mol_gru_bih']),
        p['mol_gru_whh'], row(p['mol_gru_bhh']),
        p['mol_align_w'], p['mol_align_b'].reshape(1, 1),
        p['mol_attend_w'], row(p['mol_attend_b']),
        p['out_w'], p['out_b'].reshape(1, 1),
    ]

    def data_spec(ncol):
        return pl.BlockSpec((R, ncol), lambda i: (i, 0))

    def full_spec(a):
        nd = a.ndim
        return pl.BlockSpec(a.shape, lambda i, _n=nd: (0,) * _n)

    in_specs = [data_spec(FDIM), data_spec(BDIM), data_spec(NB), data_spec(NB),
                data_spec(1)] + [full_spec(a) for a in ins[5:]]

    out_shapes = (
        jax.ShapeDtypeStruct((b * mol, FP), jnp.float32),
        jax.ShapeDtypeStruct((b, 1), jnp.float32),
    )
    out_specs = (
        pl.BlockSpec((R, FP), lambda i: (i, 0)),
        pl.BlockSpec((BS, 1), lambda i: (i, 0)),
    )

    out_af, out_pred = pl.pallas_call(
        _fp_kernel,
        grid=(n_blocks,),
        in_specs=in_specs,
        out_specs=out_specs,
        out_shape=out_shapes,
        interpret=interpret,
    )(*ins)

    return out_af.reshape(b, mol, FP), out_pred


def kernel(atom_list, bond_list, atom_degree_list, bond_degree_list,
           atom_mask, params):
    return _fingerprint(atom_list, bond_list, atom_degree_list,
                        bond_degree_list, atom_mask, params)


# speedup vs baseline: 24.9255x; 2.0908x over previous
import functools

import jax
import jax.numpy as jnp
from jax.experimental import pallas as pl

RADIUS = 2
T = 2
FDIM = 39
BDIM = 10
FP = 64
B = 1024
MOL = 128
NB = 6

BS = 8

_HI = jax.lax.Precision.DEFAULT


def _dotT(x, w):
    return jax.lax.dot_general(x, w, (((1,), (1,)), ((), ())),
                               precision=_HI, preferred_element_type=jnp.float32)


def _dot(x, w):
    return jax.lax.dot_general(x, w, (((1,), (0,)), ((), ())),
                               precision=_HI, preferred_element_type=jnp.float32)


def _seg_sum(e_mat, x):
    return jax.lax.dot_general(e_mat, x, (((0,), (0,)), ((), ())),
                               precision=_HI, preferred_element_type=jnp.float32)


def _leaky(x):
    return jnp.where(x >= 0, x, 0.01 * x)


def _elu(x):
    return jnp.where(x > 0, x, jnp.exp(jnp.minimum(x, 0.0)) - 1.0)


def _gru(x, h, wih, whh, bih, bhh):
    gi = _dotT(x, wih) + bih
    gh = _dotT(h, whh) + bhh
    r = jax.nn.sigmoid(gi[:, 0:FP] + gh[:, 0:FP])
    z = jax.nn.sigmoid(gi[:, FP:2 * FP] + gh[:, FP:2 * FP])
    n = jnp.tanh(gi[:, 2 * FP:] + r * gh[:, 2 * FP:])
    return (1.0 - z) * n + z * h


def _onehots(idx):
    iota = jax.lax.broadcasted_iota(jnp.int32, (MOL, MOL), 1)
    out = []
    for m in range(BS):
        blk = idx[m * MOL:(m + 1) * MOL, :]
        out.append(jnp.concatenate(
            [(blk[:, j:j + 1] == iota).astype(jnp.float32) for j in range(NB)],
            axis=0))
    return out


def _gather_all(oh, table):
    g = [_dot(oh[m], table[m * MOL:(m + 1) * MOL, :]) for m in range(BS)]
    return [jnp.concatenate([g[m][j * MOL:(j + 1) * MOL, :] for m in range(BS)],
                            axis=0) for j in range(NB)]


def _attention_round(af, nf_list, idxA, alw, alb, atw, atb):
    wa = alw[:, 0:FP]
    wn = alw[:, FP:2 * FP]
    base = jnp.sum(af * wa, axis=1, keepdims=True) + alb
    scores = []
    amasks = []
    for j in range(NB):
        pad = idxA[:, j:j + 1] == (MOL - 1)
        smask = jnp.where(pad, -9e8, 0.0).astype(jnp.float32)
        amasks.append(jnp.where(pad, 0.0, 1.0).astype(jnp.float32))
        scores.append(_leaky(base + jnp.sum(nf_list[j] * wn, axis=1,
                                            keepdims=True)) + smask)
    mx = scores[0]
    for j in range(1, NB):
        mx = jnp.maximum(mx, scores[j])
    es = [jnp.exp(s - mx) for s in scores]
    denom = es[0]
    for j in range(1, NB):
        denom = denom + es[j]
    ctx = jnp.zeros_like(af)
    for j in range(NB):
        aw = es[j] / denom * amasks[j]
        ctx = ctx + aw * (_dotT(nf_list[j], atw) + atb)
    return _elu(ctx)


def _fp_kernel(atom_ref, bond_ref, idxA_ref, idxB_ref, mask_ref,
               afc_w_ref, afc_b_ref, nfc_w_ref, nfc_b_ref,
               gwi_ref, gbi_ref, gwh_ref, gbh_ref,
               alw_ref, alb_ref, atw_ref, atb_ref,
               mgwi_ref, mgbi_ref, mgwh_ref, mgbh_ref,
               malw_ref, malb_ref, matw_ref, matb_ref,
               ow_ref, ob_ref,
               out_af_ref, out_pred_ref):
    R = BS * MOL
    atom = atom_ref[...]
    bond = bond_ref[...]
    idxA = idxA_ref[...]
    idxB = idxB_ref[...]
    mask = mask_ref[...]

    af = _leaky(_dotT(atom, afc_w_ref[...]) + afc_b_ref[...])

    nfc_w = nfc_w_ref[...]
    atomP = _dotT(atom, nfc_w[:, 0:FDIM])
    bondP = _dotT(bond, nfc_w[:, FDIM:FDIM + BDIM])
    ohA = _onehots(idxA)
    ohB = _onehots(idxB)
    gA = _gather_all(ohA, atomP)
    gB = _gather_all(ohB, bondP)
    nf0 = [_leaky(gA[j] + gB[j] + nfc_b_ref[...]) for j in range(NB)]

    h = af
    for d in range(RADIUS):
        if d == 0:
            nf = nf0
            cur = af
        else:
            nf = _gather_all(ohA, act)
            cur = act
        ctx = _attention_round(
            cur, nf, idxA,
            alw_ref[d], alb_ref[d], atw_ref[d], atb_ref[d])
        h = _gru(ctx, h, gwi_ref[d], gwh_ref[d], gbi_ref[d], gbh_ref[d])
        act = jnp.maximum(h, 0.0)

    out_af_ref[...] = h

    e_mat = (jax.lax.broadcasted_iota(jnp.int32, (R, BS), 0) // MOL ==
             jax.lax.broadcasted_iota(jnp.int32, (R, BS), 1)).astype(jnp.float32)
    mol_feature = _seg_sum(e_mat, act * mask)
    act_mol = jnp.maximum(mol_feature, 0.0)
    mol_smask = jnp.where(mask == 0.0, -9e8, 0.0).astype(jnp.float32)

    malw = malw_ref[...]
    w_mol = malw[:, 0:FP]
    w_atom = malw[:, FP:2 * FP]
    aft = _dotT(act, matw_ref[...]) + matb_ref[...]
    for _ in range(T):
        act_mol_rows = _dot(e_mat, act_mol)
        mas = _leaky(jnp.sum(act_mol_rows * w_mol, axis=1, keepdims=True) +
                     jnp.sum(act * w_atom, axis=1, keepdims=True) +
                     malb_ref[...]) + mol_smask
        shifted = []
        for m in range(BS):
            sl = mas[m * MOL:(m + 1) * MOL, :]
            shifted.append(sl - jnp.max(sl))
        e = jnp.exp(jnp.concatenate(shifted, axis=0)) * mask
        num = _seg_sum(e_mat, e * aft)
        den = _seg_sum(e_mat, jnp.broadcast_to(e, (R, FP)))
        mc = _elu(num / den)
        mol_feature = _gru(mc, mol_feature, mgwi_ref[...], mgwh_ref[...],
                           mgbi_ref[...], mgbh_ref[...])
        act_mol = jnp.maximum(mol_feature, 0.0)

    out_pred_ref[...] = (jnp.sum(mol_feature * ow_ref[...], axis=1,
                                 keepdims=True) + ob_ref[...])


@functools.partial(jax.jit, static_argnames=("interpret",))
def _fingerprint(atom_list, bond_list, atom_degree_list, bond_degree_list,
                 atom_mask, params, interpret=False):
    b, mol, fdim = atom_list.shape
    assert mol == MOL and fdim == FDIM
    R = BS * MOL
    n_blocks = b // BS

    atom2d = atom_list.reshape(b * mol, FDIM)
    bond2d = bond_list.reshape(b * mol, BDIM)
    idxA2d = atom_degree_list.reshape(b * mol, NB).astype(jnp.int32)
    idxB2d = bond_degree_list.reshape(b * mol, NB).astype(jnp.int32)
    mask2d = atom_mask.reshape(b * mol, 1)

    p = params
    row = lambda v: v.reshape(1, -1)
    ins = [
        atom2d, bond2d, idxA2d, idxB2d, mask2d,
        p['atom_fc_w'], row(p['atom_fc_b']),
        p['neighbor_fc_w'], row(p['neighbor_fc_b']),
        p['gru_wih'], p['gru_bih'].reshape(RADIUS, 1, 3 * FP),
        p['gru_whh'], p['gru_bhh'].reshape(RADIUS, 1, 3 * FP),
        p['align_w'], p['align_b'].reshape(RADIUS, 1, 1),
        p['attend_w'], p['attend_b'].reshape(RADIUS, 1, FP),
        p['mol_gru_wih'], row(p['mol_gru_bih']),
        p['mol_gru_whh'], row(p['mol_gru_bhh']),
        p['mol_align_w'], p['mol_align_b'].reshape(1, 1),
        p['mol_attend_w'], row(p['mol_attend_b']),
        p['out_w'], p['out_b'].reshape(1, 1),
    ]

    def data_spec(ncol):
        return pl.BlockSpec((R, ncol), lambda i: (i, 0))

    def full_spec(a):
        nd = a.ndim
        return pl.BlockSpec(a.shape, lambda i, _n=nd: (0,) * _n)

    in_specs = [data_spec(FDIM), data_spec(BDIM), data_spec(NB), data_spec(NB),
                data_spec(1)] + [full_spec(a) for a in ins[5:]]

    out_shapes = (
        jax.ShapeDtypeStruct((b * mol, FP), jnp.float32),
        jax.ShapeDtypeStruct((b, 1), jnp.float32),
    )
    out_specs = (
        pl.BlockSpec((R, FP), lambda i: (i, 0)),
        pl.BlockSpec((BS, 1), lambda i: (i, 0)),
    )

    out_af, out_pred = pl.pallas_call(
        _fp_kernel,
        grid=(n_blocks,),
        in_specs=in_specs,
        out_specs=out_specs,
        out_shape=out_shapes,
        interpret=interpret,
    )(*ins)

    return out_af.reshape(b, mol, FP), out_pred


def kernel(atom_list, bond_list, atom_degree_list, bond_degree_list,
           atom_mask, params):
    return _fingerprint(atom_list, bond_list, atom_degree_list,
                        bond_degree_list, atom_mask, params)


# speedup vs baseline: 26.1258x; 1.0482x over previous
import functools

import jax
import jax.numpy as jnp
from jax.experimental import pallas as pl

RADIUS = 2
T = 2
FDIM = 39
BDIM = 10
FP = 64
B = 1024
MOL = 128
NB = 6

BS = 16

_PREC = jax.lax.Precision.DEFAULT


def _dotT(x, w):
    return jax.lax.dot_general(x, w, (((1,), (1,)), ((), ())),
                               precision=_PREC, preferred_element_type=jnp.float32)


def _dot(x, w):
    return jax.lax.dot_general(x, w, (((1,), (0,)), ((), ())),
                               precision=_PREC, preferred_element_type=jnp.float32)


def _seg_sum(e_mat, x):
    return jax.lax.dot_general(e_mat, x, (((0,), (0,)), ((), ())),
                               precision=_PREC, preferred_element_type=jnp.float32)


def _leaky(x):
    return jnp.where(x >= 0, x, 0.01 * x)


def _elu(x):
    return jnp.where(x > 0, x, jnp.exp(jnp.minimum(x, 0.0)) - 1.0)


def _gru(x, h, wih, whh, bih, bhh):
    gi = _dotT(x, wih) + bih
    gh = _dotT(h, whh) + bhh
    r = jax.nn.sigmoid(gi[:, 0:FP] + gh[:, 0:FP])
    z = jax.nn.sigmoid(gi[:, FP:2 * FP] + gh[:, FP:2 * FP])
    n = jnp.tanh(gi[:, 2 * FP:] + r * gh[:, 2 * FP:])
    return (1.0 - z) * n + z * h


def _onehots(idx):
    iota = jax.lax.broadcasted_iota(jnp.int32, (MOL, MOL), 1)
    out = []
    for m in range(BS):
        blk = idx[m * MOL:(m + 1) * MOL, :]
        out.append(jnp.concatenate(
            [(blk[:, j:j + 1] == iota).astype(jnp.float32) for j in range(NB)],
            axis=0))
    return out


def _fp_kernel(atom_ref, bond_ref, idxA_ref, idxB_ref, mask_ref,
               afc_w_ref, afc_b_ref, nfc_w_ref, nfc_b_ref,
               gwi_ref, gbi_ref, gwh_ref, gbh_ref,
               wcomb_ref, wapad_ref, alb_ref, atb_ref,
               mgwi_ref, mgbi_ref, mgwh_ref, mgbh_ref,
               wmaspad_ref, malb_ref, matw_ref, matb_ref,
               ow_ref, ob_ref,
               out_af_ref, out_pred_ref):
    R = BS * MOL
    atom = atom_ref[...]
    bond = bond_ref[...]
    idxA = idxA_ref[...]
    idxB = idxB_ref[...]
    mask = mask_ref[...]

    af = _leaky(_dotT(atom, afc_w_ref[...]) + afc_b_ref[...])

    nfc_w = nfc_w_ref[...]
    atomP = _dotT(atom, nfc_w[:, 0:FDIM])
    bondP = _dotT(bond, nfc_w[:, FDIM:FDIM + BDIM])
    ohA = _onehots(idxA)
    ohB = _onehots(idxB)

    pad = idxA == (MOL - 1)
    smask = jnp.where(pad, -9e8, 0.0).astype(jnp.float32)
    amask = jnp.where(pad, 0.0, 1.0).astype(jnp.float32)

    h = af
    g0 = None
    for d in range(RADIUS):
        cur = af if d == 0 else act
        base = _dot(cur, wapad_ref[d])[:, 0:1] + alb_ref[d]
        wcomb = wcomb_ref[d]
        combs = []
        for m in range(BS):
            rows = slice(m * MOL, (m + 1) * MOL)
            if d == 0:
                g = _leaky(_dot(ohA[m], atomP[rows, :]) +
                           _dot(ohB[m], bondP[rows, :]) + nfc_b_ref[...])
            else:
                g = _dot(ohA[m], act[rows, :])
            combs.append(_dot(g, wcomb))
        scores = []
        for j in range(NB):
            sraw = jnp.concatenate(
                [combs[m][j * MOL:(j + 1) * MOL, FP:FP + 1] for m in range(BS)],
                axis=0)
            scores.append(_leaky(base + sraw) + smask[:, j:j + 1])
        mx = scores[0]
        for j in range(1, NB):
            mx = jnp.maximum(mx, scores[j])
        es = [jnp.exp(s - mx) for s in scores]
        denom = es[0]
        for j in range(1, NB):
            denom = denom + es[j]
        aw = [es[j] / denom * amask[:, j:j + 1] for j in range(NB)]
        swt = aw[0]
        for j in range(1, NB):
            swt = swt + aw[j]
        ctx_pieces = []
        for m in range(BS):
            rows = slice(m * MOL, (m + 1) * MOL)
            c = aw[0][rows, :] * combs[m][0:MOL, 0:FP]
            for j in range(1, NB):
                c = c + aw[j][rows, :] * combs[m][j * MOL:(j + 1) * MOL, 0:FP]
            ctx_pieces.append(c)
        ctx = _elu(jnp.concatenate(ctx_pieces, axis=0) + swt * atb_ref[d])
        h = _gru(ctx, h, gwi_ref[d], gwh_ref[d], gbi_ref[d], gbh_ref[d])
        act = jnp.maximum(h, 0.0)

    out_af_ref[...] = h

    e_mat = (jax.lax.broadcasted_iota(jnp.int32, (R, BS), 0) // MOL ==
             jax.lax.broadcasted_iota(jnp.int32, (R, BS), 1)).astype(jnp.float32)
    mol_feature = _seg_sum(e_mat, act * mask)
    act_mol = jnp.maximum(mol_feature, 0.0)
    mol_smask = jnp.where(mask == 0.0, -9e8, 0.0).astype(jnp.float32)

    wmaspad = wmaspad_ref[...]
    aft = _dotT(act, matw_ref[...]) + matb_ref[...]
    atom_score = _dot(act, wmaspad)[:, 0:1]
    for _ in range(T):
        act_mol_rows = _dot(e_mat, act_mol)
        mol_score = _dot(act_mol_rows, wmaspad)[:, 1:2]
        mas = _leaky(atom_score + mol_score + malb_ref[...]) + mol_smask
        shifted = []
        for m in range(BS):
            sl = mas[m * MOL:(m + 1) * MOL, :]
            shifted.append(sl - jnp.max(sl))
        e = jnp.exp(jnp.concatenate(shifted, axis=0)) * mask
        num = _seg_sum(e_mat, e * aft)
        den = _seg_sum(e_mat, jnp.broadcast_to(e, (R, FP)))
        mc = _elu(num / den)
        mol_feature = _gru(mc, mol_feature, mgwi_ref[...], mgwh_ref[...],
                           mgbi_ref[...], mgbh_ref[...])
        act_mol = jnp.maximum(mol_feature, 0.0)

    out_pred_ref[...] = (jnp.sum(mol_feature * ow_ref[...], axis=1,
                                 keepdims=True) + ob_ref[...])


@functools.partial(jax.jit, static_argnames=("interpret",))
def _fingerprint(atom_list, bond_list, atom_degree_list, bond_degree_list,
                 atom_mask, params, interpret=False):
    b, mol, fdim = atom_list.shape
    assert mol == MOL and fdim == FDIM
    R = BS * MOL
    n_blocks = b // BS

    atom2d = atom_list.reshape(b * mol, FDIM)
    bond2d = bond_list.reshape(b * mol, BDIM)
    idxA2d = atom_degree_list.reshape(b * mol, NB).astype(jnp.int32)
    idxB2d = bond_degree_list.reshape(b * mol, NB).astype(jnp.int32)
    mask2d = atom_mask.reshape(b * mol, 1)

    p = params
    row = lambda v: v.reshape(1, -1)

    wcomb = jnp.concatenate([
        jnp.transpose(p['attend_w'], (0, 2, 1)),
        p['align_w'][:, :, FP:2 * FP].transpose(0, 2, 1),
        jnp.zeros((RADIUS, FP, 128 - FP - 1), jnp.float32)], axis=2)
    wapad = jnp.concatenate([
        p['align_w'][:, :, 0:FP].transpose(0, 2, 1),
        jnp.zeros((RADIUS, FP, 127), jnp.float32)], axis=2)
    wmaspad = jnp.concatenate([
        p['mol_align_w'][:, FP:2 * FP].T,
        p['mol_align_w'][:, 0:FP].T,
        jnp.zeros((FP, 126), jnp.float32)], axis=1)

    ins = [
        atom2d, bond2d, idxA2d, idxB2d, mask2d,
        p['atom_fc_w'], row(p['atom_fc_b']),
        p['neighbor_fc_w'], row(p['neighbor_fc_b']),
        p['gru_wih'], p['gru_bih'].reshape(RADIUS, 1, 3 * FP),
        p['gru_whh'], p['gru_bhh'].reshape(RADIUS, 1, 3 * FP),
        wcomb, wapad,
        p['align_b'].reshape(RADIUS, 1, 1),
        p['attend_b'].reshape(RADIUS, 1, FP),
        p['mol_gru_wih'], row(p['mol_gru_bih']),
        p['mol_gru_whh'], row(p['mol_gru_bhh']),
        wmaspad, p['mol_align_b'].reshape(1, 1),
        p['mol_attend_w'], row(p['mol_attend_b']),
        p['out_w'], p['out_b'].reshape(1, 1),
    ]

    def data_spec(ncol):
        return pl.BlockSpec((R, ncol), lambda i: (i, 0))

    def full_spec(a):
        nd = a.ndim
        return pl.BlockSpec(a.shape, lambda i, _n=nd: (0,) * _n)

    in_specs = [data_spec(FDIM), data_spec(BDIM), data_spec(NB), data_spec(NB),
                data_spec(1)] + [full_spec(a) for a in ins[5:]]

    out_shapes = (
        jax.ShapeDtypeStruct((b * mol, FP), jnp.float32),
        jax.ShapeDtypeStruct((b, 1), jnp.float32),
    )
    out_specs = (
        pl.BlockSpec((R, FP), lambda i: (i, 0)),
        pl.BlockSpec((BS, 1), lambda i: (i, 0)),
    )

    out_af, out_pred = pl.pallas_call(
        _fp_kernel,
        grid=(n_blocks,),
        in_specs=in_specs,
        out_specs=out_specs,
        out_shape=out_shapes,
        interpret=interpret,
    )(*ins)

    return out_af.reshape(b, mol, FP), out_pred


def kernel(atom_list, bond_list, atom_degree_list, bond_degree_list,
           atom_mask, params):
    return _fingerprint(atom_list, bond_list, atom_degree_list,
                        bond_degree_list, atom_mask, params)


# speedup vs baseline: 61.5045x; 2.3542x over previous
import functools

import jax
import jax.numpy as jnp
from jax.experimental import pallas as pl

RADIUS = 2
T = 2
FDIM = 39
BDIM = 10
FP = 64
B = 1024
MOL = 128
NB = 6

BS = 16

_PREC = jax.lax.Precision.DEFAULT


def _dot(w, x):
    return jax.lax.dot_general(w, x, (((1,), (0,)), ((), ())),
                               precision=_PREC, preferred_element_type=jnp.float32)


def _leaky(x):
    return jnp.where(x >= 0, x, 0.01 * x)


def _elu(x):
    return jnp.where(x > 0, x, jnp.exp(jnp.minimum(x, 0.0)) - 1.0)


def _gru(x, h, wih, whh, bih, bhh):
    gi = _dot(wih, x) + bih
    gh = _dot(whh, h) + bhh
    r = jax.nn.sigmoid(gi[0:FP, :] + gh[0:FP, :])
    z = jax.nn.sigmoid(gi[FP:2 * FP, :] + gh[FP:2 * FP, :])
    n = jnp.tanh(gi[2 * FP:, :] + r * gh[2 * FP:, :])
    return (1.0 - z) * n + z * h


def _flat6(rowblk):
    return jnp.concatenate([rowblk[j:j + 1, :] for j in range(NB)], axis=1)


def _rep6(row):
    return jnp.concatenate([row] * NB, axis=1)


def _fp_kernel(atom_ref, bond_ref, idxA_ref, idxB_ref, mask_ref,
               afc_w_ref, afc_b_ref, nfcA_ref, nfcB_ref, nfc_b_ref,
               gwi_ref, gbi_ref, gwh_ref, gbh_ref,
               wcomb_ref, wa8_ref, alb_ref, atb_ref,
               mgwi_ref, mgbi_ref, mgwh_ref, mgbh_ref,
               wmas8_ref, malb_ref, matw_ref, matb_ref,
               ow8_ref, ob_ref,
               out_af_ref, out_pred_ref):
    R = BS * MOL
    atom = atom_ref[...]
    bond = bond_ref[...]
    idxA = idxA_ref[...]
    idxB = idxB_ref[...]
    mask = mask_ref[...]

    af = _leaky(_dot(afc_w_ref[...], atom) + afc_b_ref[...])

    atomP = _dot(nfcA_ref[...], atom)
    bondP = _dot(nfcB_ref[...], bond)

    iota = jax.lax.broadcasted_iota(jnp.int32, (MOL, NB * MOL), 0)
    flA, flB, ohA, ohB = [], [], [], []
    for m in range(BS):
        cols = slice(m * MOL, (m + 1) * MOL)
        fa = _flat6(idxA[:, cols])
        fb = _flat6(idxB[:, cols])
        flA.append(fa)
        flB.append(fb)
        ohA.append((jnp.broadcast_to(fa, (MOL, NB * MOL)) == iota
                    ).astype(jnp.float32))
        ohB.append((jnp.broadcast_to(fb, (MOL, NB * MOL)) == iota
                    ).astype(jnp.float32))

    h = af
    for d in range(RADIUS):
        cur = af if d == 0 else act
        base = _dot(wa8_ref[d], cur)[0:1, :] + alb_ref[d]
        wcomb = wcomb_ref[d]
        ctx_pieces = []
        swt_pieces = []
        for m in range(BS):
            cols = slice(m * MOL, (m + 1) * MOL)
            if d == 0:
                g = _leaky(_dot(atomP[:, cols], ohA[m]) +
                           _dot(bondP[:, cols], ohB[m]) + nfc_b_ref[...])
            else:
                g = _dot(act[:, cols], ohA[m])
            comb = _dot(wcomb, g)
            nft = comb[0:FP, :]
            sraw = comb[FP:FP + 1, :]
            padf = flA[m] == (MOL - 1)
            s = (_leaky(_rep6(base[:, cols]) + sraw) +
                 jnp.where(padf, -9e8, 0.0).astype(jnp.float32))
            mx = jnp.maximum(s[:, 0:MOL], s[:, MOL:2 * MOL])
            for j in range(2, NB):
                mx = jnp.maximum(mx, s[:, j * MOL:(j + 1) * MOL])
            e = jnp.exp(s - _rep6(mx))
            den = e[:, 0:MOL] + e[:, MOL:2 * MOL]
            for j in range(2, NB):
                den = den + e[:, j * MOL:(j + 1) * MOL]
            aw = (e / _rep6(den) *
                  jnp.where(padf, 0.0, 1.0).astype(jnp.float32))
            c = aw[:, 0:MOL] * nft[:, 0:MOL]
            swt = aw[:, 0:MOL]
            for j in range(1, NB):
                c = c + aw[:, j * MOL:(j + 1) * MOL] * nft[:, j * MOL:(j + 1) * MOL]
                swt = swt + aw[:, j * MOL:(j + 1) * MOL]
            ctx_pieces.append(c)
            swt_pieces.append(swt)
        ctx = jnp.concatenate(ctx_pieces, axis=1)
        swt = jnp.concatenate(swt_pieces, axis=1)
        ctx = _elu(ctx + atb_ref[...][:, d:d + 1] * swt)
        h = _gru(ctx, h, gwi_ref[d], gwh_ref[d], gbi_ref[d], gbh_ref[d])
        act = jnp.maximum(h, 0.0)

    out_af_ref[...] = h

    esel = (jax.lax.broadcasted_iota(jnp.int32, (R, MOL), 0) // MOL ==
            jax.lax.broadcasted_iota(jnp.int32, (R, MOL), 1)).astype(jnp.float32)
    mol_feature = _dot(act * mask, esel)
    act_mol = jnp.maximum(mol_feature, 0.0)
    mol_smask = jnp.where(mask == 0.0, -9e8, 0.0).astype(jnp.float32)

    wmas8 = wmas8_ref[...]
    aft = _dot(matw_ref[...], act) + matb_ref[...]
    atom_score = _dot(wmas8, act)[0:1, :]
    atom_rs = atom_score.reshape(BS, MOL)
    smask_rs = mol_smask.reshape(BS, MOL)
    mask_rs = mask.reshape(BS, MOL)
    for _ in range(T):
        mol_sc = _dot(wmas8, act_mol)
        sc_rows = jax.lax.dot_general(
            mol_sc, esel, (((1,), (1,)), ((), ())),
            precision=_PREC, preferred_element_type=jnp.float32)[1:2, :]
        mas = (_leaky(atom_rs + sc_rows.reshape(BS, MOL) + malb_ref[...]) +
               smask_rs)
        mas = mas - jnp.max(mas, axis=1, keepdims=True)
        e = jnp.exp(mas) * mask_rs
        e_norm = e / jnp.sum(e, axis=1, keepdims=True)
        num = _dot(aft * e_norm.reshape(1, R), esel)
        mc = _elu(num)
        mol_feature = _gru(mc, mol_feature, mgwi_ref[...], mgwh_ref[...],
                           mgbi_ref[...], mgbh_ref[...])
        act_mol = jnp.maximum(mol_feature, 0.0)

    pred = _dot(ow8_ref[...], mol_feature)[0:1, 0:BS] + ob_ref[...]
    out_pred_ref[...] = pred.reshape(1, 1, BS)


@functools.partial(jax.jit, static_argnames=("interpret",))
def _fingerprint(atom_list, bond_list, atom_degree_list, bond_degree_list,
                 atom_mask, params, interpret=False):
    b, mol, fdim = atom_list.shape
    assert mol == MOL and fdim == FDIM
    R = BS * MOL
    n_blocks = b // BS

    atomT = atom_list.reshape(b * mol, FDIM).T
    bondT = bond_list.reshape(b * mol, BDIM).T
    idxAT = atom_degree_list.reshape(b * mol, NB).astype(jnp.int32).T
    idxBT = bond_degree_list.reshape(b * mol, NB).astype(jnp.int32).T
    maskT = atom_mask.reshape(1, b * mol)

    p = params
    col = lambda v: v.reshape(-1, 1)

    wcomb = jnp.concatenate([
        p['attend_w'],
        p['align_w'][:, :, FP:2 * FP],
        jnp.zeros((RADIUS, 7, FP), jnp.float32)], axis=1)
    wa8 = jnp.concatenate([
        p['align_w'][:, :, 0:FP],
        jnp.zeros((RADIUS, 7, FP), jnp.float32)], axis=1)
    wmas8 = jnp.concatenate([
        p['mol_align_w'][:, FP:2 * FP],
        p['mol_align_w'][:, 0:FP],
        jnp.zeros((6, FP), jnp.float32)], axis=0)
    ow8 = jnp.concatenate([p['out_w'], jnp.zeros((7, FP), jnp.float32)], axis=0)

    ins = [
        atomT, bondT, idxAT, idxBT, maskT,
        p['atom_fc_w'], col(p['atom_fc_b']),
        p['neighbor_fc_w'][:, 0:FDIM], p['neighbor_fc_w'][:, FDIM:FDIM + BDIM],
        col(p['neighbor_fc_b']),
        p['gru_wih'], p['gru_bih'].reshape(RADIUS, 3 * FP, 1),
        p['gru_whh'], p['gru_bhh'].reshape(RADIUS, 3 * FP, 1),
        wcomb, wa8,
        p['align_b'].reshape(RADIUS, 1, 1),
        p['attend_b'].T,
        p['mol_gru_wih'], col(p['mol_gru_bih']),
        p['mol_gru_whh'], col(p['mol_gru_bhh']),
        wmas8, p['mol_align_b'].reshape(1, 1),
        p['mol_attend_w'], col(p['mol_attend_b']),
        ow8, p['out_b'].reshape(1, 1),
    ]

    def data_spec(nrow):
        return pl.BlockSpec((nrow, R), lambda i: (0, i))

    def full_spec(a):
        nd = a.ndim
        return pl.BlockSpec(a.shape, lambda i, _n=nd: (0,) * _n)

    in_specs = [data_spec(FDIM), data_spec(BDIM), data_spec(NB), data_spec(NB),
                data_spec(1)] + [full_spec(a) for a in ins[5:]]

    out_shapes = (
        jax.ShapeDtypeStruct((FP, b * mol), jnp.float32),
        jax.ShapeDtypeStruct((n_blocks, 1, BS), jnp.float32),
    )
    out_specs = (
        pl.BlockSpec((FP, R), lambda i: (0, i)),
        pl.BlockSpec((1, 1, BS), lambda i: (i, 0, 0)),
    )

    out_af, out_pred = pl.pallas_call(
        _fp_kernel,
        grid=(n_blocks,),
        in_specs=in_specs,
        out_specs=out_specs,
        out_shape=out_shapes,
        interpret=interpret,
    )(*ins)

    return (out_af.T.reshape(b, mol, FP),
            out_pred.reshape(b, 1))


def kernel(atom_list, bond_list, atom_degree_list, bond_degree_list,
           atom_mask, params):
    return _fingerprint(atom_list, bond_list, atom_degree_list,
                        bond_degree_list, atom_mask, params)
